# baseline (speedup 1.0000x reference)
import jax
import jax.numpy as jnp
from jax import lax
from jax.experimental import pallas as pl
from jax.experimental.pallas import tpu as pltpu
from jax.experimental.pallas import tpu_sc as plsc

D_MODEL = 2048
SEQ_LEN = 4096
BATCH = 4
ROWS = SEQ_LEN * BATCH
NUM_WORKERS = 32
ROWS_PER_WORKER = ROWS // NUM_WORKERS
CHUNK = 16
NUM_CHUNKS = ROWS_PER_WORKER // CHUNK
LANES = 16


def _sc_body(x_hbm, idx_hbm, pe_hbm, out_hbm, idx_v, pos_v, buf, sem):
    cid = lax.axis_index("c")
    sid = lax.axis_index("s")
    wid = sid * 2 + cid
    row0 = wid * ROWS_PER_WORKER

    pltpu.sync_copy(idx_hbm, idx_v)
    pltpu.async_copy(pe_hbm.at[idx_v], pos_v, sem).wait()

    def chunk_body(c, carry):
        r0 = row0 + c * CHUNK
        pltpu.sync_copy(x_hbm.at[pl.ds(r0, CHUNK)], buf)

        def col_body(k, carry2):
            o = k * LANES
            for j in range(CHUNK):
                plsc.addupdate(
                    buf.at[j, pl.ds(o, LANES)],
                    pos_v[j, pl.ds(o, LANES)],
                )
            return carry2

        lax.fori_loop(0, D_MODEL // LANES, col_body, 0)
        pltpu.sync_copy(buf, out_hbm.at[pl.ds(r0, CHUNK)])
        return carry

    lax.fori_loop(0, NUM_CHUNKS, chunk_body, 0)


def kernel(x, pe, idx):
    xf = x.reshape(ROWS, D_MODEL)
    idx16 = jnp.tile(idx.astype(jnp.int32), CHUNK // BATCH)
    out = pl.kernel(
        _sc_body,
        out_type=jax.ShapeDtypeStruct((ROWS, D_MODEL), jnp.float32),
        mesh=plsc.VectorSubcoreMesh(core_axis_name="c", subcore_axis_name="s"),
        scratch_types=[
            pltpu.VMEM((CHUNK,), jnp.int32),
            pltpu.VMEM((CHUNK, D_MODEL), jnp.float32),
            pltpu.VMEM((CHUNK, D_MODEL), jnp.float32),
            pltpu.SemaphoreType.DMA,
        ],
    )(xf, idx16, pe)
    return out.reshape(SEQ_LEN, BATCH, D_MODEL)


# speedup vs baseline: 1.4507x; 1.4507x over previous
import jax
import jax.numpy as jnp
from jax import lax
from jax.experimental import pallas as pl
from jax.experimental.pallas import tpu as pltpu
from jax.experimental.pallas import tpu_sc as plsc

D_MODEL = 2048
SEQ_LEN = 4096
BATCH = 4
ROWS = SEQ_LEN * BATCH
NUM_WORKERS = 32
ROWS_PER_WORKER = ROWS // NUM_WORKERS
CHUNK = 16
NUM_CHUNKS = ROWS_PER_WORKER // CHUNK
NBUF = 3
POS_ROWS = 8
LANES = 16


def _sc_body(x_hbm, idx_hbm, pe_hbm, out_hbm,
             idx_v, pos_v, b0, b1, b2,
             gsem, i0, i1, i2, o0, o1, o2):
    bufs = (b0, b1, b2)
    isems = (i0, i1, i2)
    osems = (o0, o1, o2)
    cid = lax.axis_index("c")
    sid = lax.axis_index("s")
    wid = sid * 2 + cid
    row0 = wid * ROWS_PER_WORKER

    pltpu.sync_copy(idx_hbm, idx_v)
    pltpu.async_copy(pe_hbm.at[idx_v.at[pl.ds(0, POS_ROWS)]], pos_v, gsem).wait()

    def in_copy(c):
        b = c % NBUF
        return pltpu.make_async_copy(
            x_hbm.at[pl.ds(row0 + c * CHUNK, CHUNK)], bufs[b], isems[b])

    def out_copy(c):
        b = c % NBUF
        return pltpu.make_async_copy(
            bufs[b], out_hbm.at[pl.ds(row0 + c * CHUNK, CHUNK)], osems[b])

    def compute(buf):
        def col_body(k, carry):
            o = k * LANES
            for j in range(POS_ROWS):
                p = pos_v[j, pl.ds(o, LANES)]
                plsc.addupdate(buf.at[j, pl.ds(o, LANES)], p)
                plsc.addupdate(buf.at[POS_ROWS + j, pl.ds(o, LANES)], p)
            return carry
        lax.fori_loop(0, D_MODEL // LANES, col_body, 0)

    in_copy(0).start()
    for c in range(NUM_CHUNKS):
        if c + 1 < NUM_CHUNKS:
            if c >= 2:
                out_copy(c - 2).wait()
            in_copy(c + 1).start()
        in_copy(c).wait()
        compute(bufs[c % NBUF])
        out_copy(c).start()
    for c in range(NUM_CHUNKS - NBUF, NUM_CHUNKS):
        out_copy(c).wait()


def kernel(x, pe, idx):
    xf = x.reshape(ROWS, D_MODEL)
    idx16 = jnp.tile(idx.astype(jnp.int32), 4)
    out = pl.kernel(
        _sc_body,
        out_type=jax.ShapeDtypeStruct((ROWS, D_MODEL), jnp.float32),
        mesh=plsc.VectorSubcoreMesh(core_axis_name="c", subcore_axis_name="s"),
        scratch_types=[
            pltpu.VMEM((16,), jnp.int32),
            pltpu.VMEM((POS_ROWS, D_MODEL), jnp.float32),
            pltpu.VMEM((CHUNK, D_MODEL), jnp.float32),
            pltpu.VMEM((CHUNK, D_MODEL), jnp.float32),
            pltpu.VMEM((CHUNK, D_MODEL), jnp.float32),
            pltpu.SemaphoreType.DMA,
            pltpu.SemaphoreType.DMA,
            pltpu.SemaphoreType.DMA,
            pltpu.SemaphoreType.DMA,
            pltpu.SemaphoreType.DMA,
            pltpu.SemaphoreType.DMA,
            pltpu.SemaphoreType.DMA,
        ],
    )(xf, idx16, pe)
    return out.reshape(SEQ_LEN, BATCH, D_MODEL)


# speedup vs baseline: 5.1478x; 3.5486x over previous
import jax
import jax.numpy as jnp
from jax import lax
from jax.experimental import pallas as pl
from jax.experimental.pallas import tpu as pltpu
from jax.experimental.pallas import tpu_sc as plsc

D_MODEL = 2048
SEQ_LEN = 4096
BATCH = 4
NUM_WORKERS = 32
SEQ_PER_WORKER = SEQ_LEN // NUM_WORKERS
CHUNK_S = 4
NUM_CHUNKS = SEQ_PER_WORKER // CHUNK_S
NBUF = 3
POS_ROWS = 8
LANES = 16


def _sc_body(x_hbm, idx_hbm, pe_hbm, out_hbm,
             idx_v, pos_v, b0, b1, b2,
             gsem, i0, i1, i2, o0, o1, o2):
    bufs = (b0, b1, b2)
    isems = (i0, i1, i2)
    osems = (o0, o1, o2)
    cid = lax.axis_index("c")
    sid = lax.axis_index("s")
    wid = sid * 2 + cid
    seq0 = wid * SEQ_PER_WORKER

    pltpu.sync_copy(idx_hbm, idx_v)
    pltpu.async_copy(pe_hbm.at[idx_v.at[pl.ds(0, POS_ROWS)]], pos_v, gsem).wait()

    def in_copy(c):
        b = c % NBUF
        return pltpu.make_async_copy(
            x_hbm.at[pl.ds(seq0 + c * CHUNK_S, CHUNK_S)], bufs[b], isems[b])

    def out_copy(c):
        b = c % NBUF
        return pltpu.make_async_copy(
            bufs[b], out_hbm.at[pl.ds(seq0 + c * CHUNK_S, CHUNK_S)], osems[b])

    def compute(buf):
        def col_body(k, carry):
            o = k * LANES
            for bb in range(BATCH):
                p = pos_v[bb, pl.ds(o, LANES)]
                for si in range(CHUNK_S):
                    plsc.addupdate(buf.at[si, bb, pl.ds(o, LANES)], p)
            return carry
        lax.fori_loop(0, D_MODEL // LANES, col_body, 0)

    in_copy(0).start()
    for c in range(NUM_CHUNKS):
        if c + 1 < NUM_CHUNKS:
            if c >= 2:
                out_copy(c - 2).wait()
            in_copy(c + 1).start()
        in_copy(c).wait()
        compute(bufs[c % NBUF])
        out_copy(c).start()
    for c in range(NUM_CHUNKS - NBUF, NUM_CHUNKS):
        out_copy(c).wait()


def kernel(x, pe, idx):
    idx16 = jnp.tile(idx.astype(jnp.int32), 4)
    return pl.kernel(
        _sc_body,
        out_type=jax.ShapeDtypeStruct((SEQ_LEN, BATCH, D_MODEL), jnp.float32),
        mesh=plsc.VectorSubcoreMesh(core_axis_name="c", subcore_axis_name="s"),
        scratch_types=[
            pltpu.VMEM((16,), jnp.int32),
            pltpu.VMEM((POS_ROWS, D_MODEL), jnp.float32),
            pltpu.VMEM((CHUNK_S, BATCH, D_MODEL), jnp.float32),
            pltpu.VMEM((CHUNK_S, BATCH, D_MODEL), jnp.float32),
            pltpu.VMEM((CHUNK_S, BATCH, D_MODEL), jnp.float32),
            pltpu.SemaphoreType.DMA,
            pltpu.SemaphoreType.DMA,
            pltpu.SemaphoreType.DMA,
            pltpu.SemaphoreType.DMA,
            pltpu.SemaphoreType.DMA,
            pltpu.SemaphoreType.DMA,
            pltpu.SemaphoreType.DMA,
        ],
    )(x, idx16, pe)


# speedup vs baseline: 5.7145x; 1.1101x over previous
import jax
import jax.numpy as jnp
from jax import lax
from jax.experimental import pallas as pl
from jax.experimental.pallas import tpu as pltpu
from jax.experimental.pallas import tpu_sc as plsc

D_MODEL = 2048
SEQ_LEN = 4096
BATCH = 4
POS_ROWS = 8
S_BLK = 64


def _gather_body(idx_hbm, pe_hbm, pos_hbm, idx_v, pos_v, gsem):
    cid = lax.axis_index("c")
    sid = lax.axis_index("s")

    @pl.when(jnp.logical_and(cid == 0, sid == 0))
    def _():
        pltpu.sync_copy(idx_hbm, idx_v)
        pltpu.async_copy(
            pe_hbm.at[idx_v.at[pl.ds(0, POS_ROWS)]], pos_v, gsem).wait()
        pltpu.sync_copy(pos_v, pos_hbm)


def _sc_gather(idx16, pe):
    return pl.kernel(
        _gather_body,
        out_type=jax.ShapeDtypeStruct((POS_ROWS, D_MODEL), jnp.float32),
        mesh=plsc.VectorSubcoreMesh(core_axis_name="c", subcore_axis_name="s"),
        scratch_types=[
            pltpu.VMEM((16,), jnp.int32),
            pltpu.VMEM((POS_ROWS, D_MODEL), jnp.float32),
            pltpu.SemaphoreType.DMA,
        ],
    )(idx16, pe)


def _add_body(x_ref, pos_ref, o_ref):
    pos = pos_ref[0:BATCH, :]
    o_ref[...] = x_ref[...] + pos[None, :, :]


def kernel(x, pe, idx):
    idx16 = jnp.tile(idx.astype(jnp.int32), 4)
    pos = _sc_gather(idx16, pe)
    return pl.pallas_call(
        _add_body,
        grid=(SEQ_LEN // S_BLK,),
        in_specs=[
            pl.BlockSpec((S_BLK, BATCH, D_MODEL), lambda i: (i, 0, 0)),
            pl.BlockSpec((POS_ROWS, D_MODEL), lambda i: (0, 0)),
        ],
        out_specs=pl.BlockSpec((S_BLK, BATCH, D_MODEL), lambda i: (i, 0, 0)),
        out_shape=jax.ShapeDtypeStruct((SEQ_LEN, BATCH, D_MODEL), jnp.float32),
    )(x, pos)
